# baseline (speedup 1.0000x reference)
import functools
import math

import jax
import jax.numpy as jnp
from jax import lax
from jax.experimental import pallas as pl
from jax.experimental.pallas import tpu as pltpu
from jax.experimental.pallas import tpu_sc as plsc


def _proj_kernel(x_ref, wq_ref, wkv_ref, q_ref, xkv_ref, *, S, C):
    f32 = jnp.float32
    xb = x_ref[0]
    q_ref[0] = jnp.dot(xb, wq_ref[...], preferred_element_type=f32)
    xkv_ref[0] = jnp.dot(xb, wkv_ref[...],
                         preferred_element_type=f32).astype(jnp.bfloat16)


def _attn_kernel(g_ref, q_ref, wgt_ref, ali_ref, dst_ref,
                 wx_ref, wp_ref, bp_ref, out_ref, *, S, C, H, K):
    d = C // H
    f32 = jnp.float32
    i32 = jnp.int32
    M = K * H
    G = g_ref[0]
    q = q_ref[0]

    alib = ali_ref[0]
    dstb = dst_ref[0]
    wgtb = wgt_ref[0] + 1e-6

    bd = (lax.broadcasted_iota(i32, (C, H), 0) // d ==
          lax.broadcasted_iota(i32, (C, H), 1)).astype(f32)
    bdT = bd.T

    bdw = jnp.concatenate([bd * wx_ref[0:1, :].T, bd * wx_ref[1:2, :].T],
                          axis=1)
    cc12 = jnp.dot(q, bdw, preferred_element_type=f32)
    c1 = cc12[:, :H]
    c2 = cc12[:, H:]

    kq64 = jnp.concatenate([
        jnp.dot(G[k * S:(k + 1) * S, :C] * q, bd, preferred_element_type=f32)
        for k in range(K)
    ], axis=1)

    iota_m = lax.broadcasted_iota(i32, (S, M), 1)
    rep_k = iota_m // H
    rep_h = iota_m % H
    adw = jnp.concatenate(
        [alib, dstb, wgtb, jnp.zeros((S, 128 - 3 * K), f32)], axis=1)
    ali64 = jnp.take_along_axis(adw, rep_k, axis=1)
    dst64 = jnp.take_along_axis(adw, K + rep_k, axis=1)
    wgt64 = jnp.take_along_axis(adw, 2 * K + rep_k, axis=1)
    cc = jnp.concatenate(
        [c1, c2, jnp.zeros((S, 128 - 2 * H), f32)], axis=1)
    c164 = jnp.take_along_axis(cc, rep_h, axis=1)
    c264 = jnp.take_along_axis(cc, H + rep_h, axis=1)

    scale = 1.0 / math.sqrt(d)
    p64 = jnp.exp((kq64 + ali64 * c164 + dst64 * c264) * scale) * wgt64

    redH = (lax.broadcasted_iota(i32, (M, H), 0) % H ==
            lax.broadcasted_iota(i32, (M, H), 1)).astype(f32)
    den = jnp.dot(p64, redH, preferred_element_type=f32)
    pa = jnp.dot(p64 * ali64, redH, preferred_element_type=f32)
    pd = jnp.dot(p64 * dst64, redH, preferred_element_type=f32)

    terms = [
        jnp.dot(p64[:, H * k:H * (k + 1)], bdT,
                preferred_element_type=f32) * G[k * S:(k + 1) * S, C:]
        for k in range(K)
    ]
    while len(terms) > 1:
        terms = [a + b for a, b in zip(terms[::2], terms[1::2])]
    num = terms[0]
    num = num + jnp.dot(pa, bdT, preferred_element_type=f32) * wx_ref[2:3, :]
    num = num + jnp.dot(pd, bdT, preferred_element_type=f32) * wx_ref[3:4, :]

    out = num / jnp.dot(den, bdT, preferred_element_type=f32)
    out = jnp.dot(out, wp_ref[...], preferred_element_type=f32) + bp_ref[0:1, :]
    out_ref[0] = out


def _sc_gather(table, gidx, N, D, window):
    mesh = plsc.VectorSubcoreMesh(core_axis_name="core",
                                  subcore_axis_name="subcore")

    @pl.kernel(out_type=jax.ShapeDtypeStruct((N, D), table.dtype), mesh=mesh)
    def sc_kernel(x_hbm, i_hbm, o_hbm):
        def body(i_vmem, o_vmem):
            pltpu.sync_copy(x_hbm.at[i_vmem.at[0]], o_vmem)

        pltpu.emit_pipeline(
            body,
            grid=(N // window,),
            in_specs=[pl.BlockSpec((1, window), lambda i: (0, i))],
            out_specs=[pl.BlockSpec((window, D), lambda i: (i, 0))],
            core_axis_name=("core", "subcore"),
            dimension_semantics=(pltpu.PARALLEL,),
        )(i_hbm, o_hbm)

    return sc_kernel(table, gidx)


def kernel(x, spatial_idx, spatial_wgt, alignment, dist, Wq, Wk, Wv, Wp, bp):
    B, S, T, C = x.shape
    K = spatial_idx.shape[-1]
    H = 4
    BT = B * T
    D = 2 * C
    f32 = jnp.float32

    x_ = jnp.transpose(x, (0, 2, 1, 3)).reshape(BT, S, C)
    idx = spatial_idx.reshape(BT, S, K).astype(jnp.int32)
    wgt = spatial_wgt.reshape(BT, S, K)
    ali = alignment.reshape(BT, S, K)
    dst = dist.reshape(BT, S, K)

    wx = jnp.concatenate([Wk[C:C + 2], Wv[C:C + 2],
                          jnp.zeros((4, C), f32)], axis=0)
    bp_pad = jnp.concatenate([bp.reshape(1, C), jnp.zeros((7, C), f32)], axis=0)

    q, xkv = pl.pallas_call(
        functools.partial(_proj_kernel, S=S, C=C),
        grid=(BT,),
        in_specs=[
            pl.BlockSpec((1, S, C), lambda i: (i, 0, 0)),
            pl.BlockSpec((C, C), lambda i: (0, 0)),
            pl.BlockSpec((C, D), lambda i: (0, 0)),
        ],
        out_specs=[
            pl.BlockSpec((1, S, C), lambda i: (i, 0, 0)),
            pl.BlockSpec((1, S, D), lambda i: (i, 0, 0)),
        ],
        out_shape=[
            jax.ShapeDtypeStruct((BT, S, C), f32),
            jax.ShapeDtypeStruct((BT, S, D), jnp.bfloat16),
        ],
    )(x_, Wq, jnp.concatenate([Wk[:C], Wv[:C]], axis=1))

    N = BT * K * S
    gidx = (jnp.arange(BT, dtype=jnp.int32)[:, None, None] * S
            + jnp.transpose(idx, (0, 2, 1))).reshape(1, N)
    table32 = jax.lax.bitcast_convert_type(
        xkv.reshape(BT * S, D // 2, 2), jnp.int32)
    g32 = _sc_gather(table32, gidx, N, D // 2, 128)
    gathered = jax.lax.bitcast_convert_type(
        g32.reshape(N, D // 2, 1), jnp.bfloat16).reshape(N, D)

    out = pl.pallas_call(
        functools.partial(_attn_kernel, S=S, C=C, H=H, K=K),
        grid=(BT,),
        in_specs=[
            pl.BlockSpec((1, K * S, D), lambda i: (i, 0, 0)),
            pl.BlockSpec((1, S, C), lambda i: (i, 0, 0)),
            pl.BlockSpec((1, S, K), lambda i: (i, 0, 0)),
            pl.BlockSpec((1, S, K), lambda i: (i, 0, 0)),
            pl.BlockSpec((1, S, K), lambda i: (i, 0, 0)),
            pl.BlockSpec((8, C), lambda i: (0, 0)),
            pl.BlockSpec((C, C), lambda i: (0, 0)),
            pl.BlockSpec((8, C), lambda i: (0, 0)),
        ],
        out_specs=pl.BlockSpec((1, S, C), lambda i: (i, 0, 0)),
        out_shape=jax.ShapeDtypeStruct((BT, S, C), f32),
    )(gathered.reshape(BT, K * S, D), q, wgt, ali, dst, wx, Wp, bp_pad)

    return out.reshape(B, T, S, C).transpose(0, 2, 1, 3)


# speedup vs baseline: 6.6468x; 6.6468x over previous
import functools
import math

import jax
import jax.numpy as jnp
from jax import lax
from jax.experimental import pallas as pl


def _attn_kernel(x_ref, idx_ref, wgt_ref, ali_ref, dst_ref,
                 wq_ref, wkv_ref, wx_ref, wp_ref, bp_ref,
                 out_ref, *, S, C, H, K):
    d = C // H
    f32 = jnp.float32
    bf16 = jnp.bfloat16
    i32 = jnp.int32
    M = K * H
    xb = x_ref[0]
    q = jnp.dot(xb, wq_ref[...], preferred_element_type=f32)
    xkv = jnp.dot(xb, wkv_ref[...],
                  preferred_element_type=f32).astype(bf16)

    idxb = idx_ref[0]
    alib = ali_ref[0]
    dstb = dst_ref[0]
    wgtb = wgt_ref[0] + 1e-6

    iota_j = lax.broadcasted_iota(i32, (S, S), 1)
    e_blocks = [(idxb[:, k:k + 1] == iota_j).astype(bf16) for k in range(K)]
    E = jnp.concatenate(e_blocks, axis=0)
    G = jnp.dot(E, xkv, preferred_element_type=f32).astype(bf16)

    bd = (lax.broadcasted_iota(i32, (C, H), 0) // d ==
          lax.broadcasted_iota(i32, (C, H), 1)).astype(f32)
    bdT = bd.T

    bdw = jnp.concatenate([bd * wx_ref[0:1, :].T, bd * wx_ref[1:2, :].T],
                          axis=1)
    cc12 = jnp.dot(q, bdw, preferred_element_type=f32)
    c1 = cc12[:, :H]
    c2 = cc12[:, H:]

    kq64 = jnp.concatenate([
        jnp.dot(G[k * S:(k + 1) * S, :C] * q, bd, preferred_element_type=f32)
        for k in range(K)
    ], axis=1)

    iota_m = lax.broadcasted_iota(i32, (S, M), 1)
    rep_k = iota_m // H
    rep_h = iota_m % H
    adw = jnp.concatenate(
        [alib, dstb, wgtb, jnp.zeros((S, 128 - 3 * K), f32)], axis=1)
    ali64 = jnp.take_along_axis(adw, rep_k, axis=1)
    dst64 = jnp.take_along_axis(adw, K + rep_k, axis=1)
    wgt64 = jnp.take_along_axis(adw, 2 * K + rep_k, axis=1)
    cc = jnp.concatenate(
        [c1, c2, jnp.zeros((S, 128 - 2 * H), f32)], axis=1)
    c164 = jnp.take_along_axis(cc, rep_h, axis=1)
    c264 = jnp.take_along_axis(cc, H + rep_h, axis=1)

    scale = 1.0 / math.sqrt(d)
    p64 = jnp.exp((kq64 + ali64 * c164 + dst64 * c264) * scale) * wgt64

    redH = (lax.broadcasted_iota(i32, (M, H), 0) % H ==
            lax.broadcasted_iota(i32, (M, H), 1)).astype(f32)
    den = jnp.dot(p64, redH, preferred_element_type=f32)
    pa = jnp.dot(p64 * ali64, redH, preferred_element_type=f32)
    pd = jnp.dot(p64 * dst64, redH, preferred_element_type=f32)

    terms = [
        jnp.dot(p64[:, H * k:H * (k + 1)], bdT,
                preferred_element_type=f32) * G[k * S:(k + 1) * S, C:]
        for k in range(K)
    ]
    while len(terms) > 1:
        terms = [a + b for a, b in zip(terms[::2], terms[1::2])]
    num = terms[0]
    num = num + jnp.dot(pa, bdT, preferred_element_type=f32) * wx_ref[2:3, :]
    num = num + jnp.dot(pd, bdT, preferred_element_type=f32) * wx_ref[3:4, :]

    out = num / jnp.dot(den, bdT, preferred_element_type=f32)
    out = jnp.dot(out, wp_ref[...], preferred_element_type=f32) + bp_ref[0:1, :]
    out_ref[0] = out


def kernel(x, spatial_idx, spatial_wgt, alignment, dist, Wq, Wk, Wv, Wp, bp):
    B, S, T, C = x.shape
    K = spatial_idx.shape[-1]
    H = 4
    BT = B * T
    f32 = jnp.float32

    x_ = jnp.transpose(x, (0, 2, 1, 3)).reshape(BT, S, C)
    idx = spatial_idx.reshape(BT, S, K).astype(jnp.int32)
    wgt = spatial_wgt.reshape(BT, S, K)
    ali = alignment.reshape(BT, S, K)
    dst = dist.reshape(BT, S, K)

    wx = jnp.concatenate([Wk[C:C + 2], Wv[C:C + 2],
                          jnp.zeros((4, C), f32)], axis=0)
    bp_pad = jnp.concatenate([bp.reshape(1, C), jnp.zeros((7, C), f32)], axis=0)

    grid = (BT,)
    bspec_bt = lambda: pl.BlockSpec((1, S, C), lambda i: (i, 0, 0))
    bspec_sk = lambda: pl.BlockSpec((1, S, K), lambda i: (i, 0, 0))
    bspec_w = lambda shape: pl.BlockSpec(shape, lambda i: (0, 0))

    out = pl.pallas_call(
        functools.partial(_attn_kernel, S=S, C=C, H=H, K=K),
        grid=grid,
        in_specs=[
            bspec_bt(),
            bspec_sk(),
            bspec_sk(),
            bspec_sk(),
            bspec_sk(),
            bspec_w((C, C)),
            bspec_w((C, 2 * C)),
            bspec_w((8, C)),
            bspec_w((C, C)),
            bspec_w((8, C)),
        ],
        out_specs=bspec_bt(),
        out_shape=jax.ShapeDtypeStruct((BT, S, C), f32),
    )(x_, idx, wgt, ali, dst, Wq, jnp.concatenate([Wk[:C], Wv[:C]], axis=1),
      wx, Wp, bp_pad)

    return out.reshape(B, T, S, C).transpose(0, 2, 1, 3)
